# baseline (speedup 1.0000x reference)
import jax
import jax.numpy as jnp
from jax.experimental import pallas as pl

B = 8
NUM_SLOTS = 8192
SLOT_DIM = 64
N_HEADS = 8
TOPK = 16
BOTTLENECK = 64
HALF = NUM_SLOTS // 2
LANES = 2 * SLOT_DIM

_SQRT2 = 1.4142135623730951
_HI = jax.lax.Precision.HIGHEST


def _dot(a, b, dims, precision=None):
    return jax.lax.dot_general(a, b, (dims, ((), ())), precision=precision,
                               preferred_element_type=jnp.float32)


def _body(mem_ref, keys_ref, vals_ref, erase_ref, addg_ref, beta_ref,
          w1_ref, b1_ref, w2_ref, b2_ref, decay_ref, age_ref,
          newmem_ref, w_out_ref):
    b = pl.program_id(0)
    mem = mem_ref[0]
    keys = keys_ref[b]
    vals = vals_ref[b]
    erase = erase_ref[b]
    addg = addg_ref[b]
    beta = beta_ref[b]

    rP = jax.lax.broadcasted_iota(jnp.int32, (LANES, 2), 0)
    cP = jax.lax.broadcasted_iota(jnp.int32, (LANES, 2), 1)
    P = ((rP < SLOT_DIM) == (cP == 0)).astype(jnp.float32)
    rQ = jax.lax.broadcasted_iota(jnp.int32, (2, LANES), 0)
    cQ = jax.lax.broadcasted_iota(jnp.int32, (2, LANES), 1)
    Q = ((rQ == 0) == (cQ < SLOT_DIM)).astype(jnp.float32)
    z8 = jnp.zeros((N_HEADS, SLOT_DIM), jnp.float32)

    h = _dot(vals, w1_ref[...], ((1,), (0,))) + b1_ref[...]
    h = 0.5 * h * (1.0 + jax.lax.erf(h / _SQRT2))
    cv = _dot(h, w2_ref[...], ((1,), (0,))) + b2_ref[...]
    cvg = cv * (addg * (1.0 / N_HEADS))

    kn = keys / jnp.maximum(
        jnp.sqrt(jnp.sum(keys * keys, axis=1, keepdims=True)), 1e-12)
    s_mem = _dot(mem * mem, P, ((1,), (0,)), _HI)
    inv_mem = 1.0 / jnp.maximum(jnp.sqrt(s_mem), 1e-12)
    mem_n = mem * _dot(inv_mem, Q, ((1,), (0,)), _HI)

    kn_blk = jnp.concatenate(
        [jnp.concatenate([kn, z8], axis=1),
         jnp.concatenate([z8, kn], axis=1)], axis=0)
    sim = _dot(kn_blk, mem_n, ((1,), (1,)))

    a1 = age_ref[...] + 1.0
    ab = a1 * (1.0 / (jnp.max(a1) + 1e-8))
    ab16 = jnp.concatenate(
        [jnp.broadcast_to(ab[0:1], (N_HEADS, HALF)),
         jnp.broadcast_to(ab[1:2], (N_HEADS, HALF))], axis=0)
    beta16 = jnp.concatenate([beta, beta], axis=0)
    sim = sim * beta16 + ab16

    colI = jax.lax.broadcasted_iota(jnp.int32, (2 * N_HEADS, HALF), 1)
    rowI = jax.lax.broadcasted_iota(jnp.int32, (2 * N_HEADS, HALF), 0)
    idxg = 2 * colI + (rowI >= N_HEADS).astype(jnp.int32)

    def pair(x, comb):
        y = comb(x[:N_HEADS], x[N_HEADS:])
        return jnp.concatenate([y, y], axis=0)

    work = sim
    mask = jnp.zeros((2 * N_HEADS, HALF), dtype=jnp.bool_)
    m0 = pair(jnp.max(work, axis=1, keepdims=True), jnp.maximum)
    for _ in range(TOPK):
        m = pair(jnp.max(work, axis=1, keepdims=True), jnp.maximum)
        cand = jnp.where(work == m, idxg, NUM_SLOTS)
        first = pair(jnp.min(cand, axis=1, keepdims=True), jnp.minimum)
        sel = idxg == first
        mask = jnp.logical_or(mask, sel)
        work = jnp.where(sel, -jnp.inf, work)

    wexp = jnp.where(mask, jnp.exp(sim - m0), 0.0)
    denom = pair(jnp.sum(wexp, axis=1, keepdims=True), jnp.add)
    w2 = wexp / denom
    w_out_ref[0] = w2

    er = erase * (1.0 / N_HEADS)
    z1 = jnp.zeros((N_HEADS, 1), jnp.float32)
    er16 = jnp.concatenate(
        [jnp.concatenate([er, z1], axis=0),
         jnp.concatenate([z1, er], axis=0)], axis=1)
    e_pair = _dot(w2, er16, ((0,), (0,)), _HI)
    cvg_blk = jnp.concatenate(
        [jnp.concatenate([cvg, z8], axis=1),
         jnp.concatenate([z8, cvg], axis=1)], axis=0)
    a_pack = _dot(w2, cvg_blk, ((0,), (0,)), _HI)

    e_b = _dot(e_pair, Q, ((1,), (0,)), _HI)
    new = mem - mem * e_b + a_pack + 1e-8
    s_new = _dot(new * new, P, ((1,), (0,)), _HI)
    dec = jax.nn.sigmoid(decay_ref[...])
    scale = dec / jnp.maximum(jnp.sqrt(s_new), 1e-12)
    newmem_ref[0] = new * _dot(scale, Q, ((1,), (0,)), _HI)


@jax.jit
def kernel(memory, write_keys, write_vals, erase, add_gate, beta,
           W1, b1, W2, b2, decay_gate, age):
    full = lambda s: pl.BlockSpec(s, lambda b: tuple(0 for _ in s))
    grid_spec = pl.GridSpec(
        grid=(B,),
        in_specs=[
            pl.BlockSpec((1, HALF, LANES), lambda b: (b, 0, 0)),
            full((B, N_HEADS, SLOT_DIM)),
            full((B, N_HEADS, SLOT_DIM)),
            full((B, N_HEADS, 1)),
            full((B, N_HEADS, 1)),
            full((B, N_HEADS, 1)),
            full((SLOT_DIM, BOTTLENECK)),
            full((1, BOTTLENECK)),
            full((BOTTLENECK, SLOT_DIM)),
            full((1, SLOT_DIM)),
            full((HALF, 2)),
            full((2, HALF)),
        ],
        out_specs=[
            pl.BlockSpec((1, HALF, LANES), lambda b: (b, 0, 0)),
            pl.BlockSpec((1, 2 * N_HEADS, HALF), lambda b: (b, 0, 0)),
        ],
    )
    nm2, w2 = pl.pallas_call(
        _body,
        grid_spec=grid_spec,
        out_shape=[
            jax.ShapeDtypeStruct((B, HALF, LANES), jnp.float32),
            jax.ShapeDtypeStruct((B, 2 * N_HEADS, HALF), jnp.float32),
        ],
    )(memory.reshape(B, HALF, LANES), write_keys, write_vals,
      erase[..., None], add_gate[..., None], beta[..., None],
      W1, b1.reshape(1, BOTTLENECK), W2, b2.reshape(1, SLOT_DIM),
      decay_gate.reshape(HALF, 2), jnp.transpose(age.reshape(HALF, 2)))
    new_memory = nm2.reshape(B, NUM_SLOTS, SLOT_DIM)
    weights = (w2.reshape(B, 2, N_HEADS, HALF)
               .transpose(0, 2, 3, 1).reshape(B, N_HEADS, NUM_SLOTS))
    return (new_memory, weights)


# speedup vs baseline: 1.3725x; 1.3725x over previous
import jax
import jax.numpy as jnp
from jax.experimental import pallas as pl

B = 8
NUM_SLOTS = 8192
SLOT_DIM = 64
N_HEADS = 8
TOPK = 16
BOTTLENECK = 64
HALF = NUM_SLOTS // 2
LANES = 2 * SLOT_DIM

_SQRT2 = 1.4142135623730951
_HI = jax.lax.Precision.HIGHEST


def _dot(a, b, dims, precision=None):
    return jax.lax.dot_general(a, b, (dims, ((), ())), precision=precision,
                               preferred_element_type=jnp.float32)


def _body(mem_ref, keys_ref, vals_ref, erase_ref, addg_ref, beta_ref,
          w1_ref, b1_ref, w2_ref, b2_ref, decay_ref, age_ref,
          newmem_ref, w_out_ref):
    b = pl.program_id(0)
    mem = mem_ref[0]
    keys = keys_ref[b]
    vals = vals_ref[b]
    erase = erase_ref[b]
    addg = addg_ref[b]
    beta = beta_ref[b]

    rP = jax.lax.broadcasted_iota(jnp.int32, (LANES, 2), 0)
    cP = jax.lax.broadcasted_iota(jnp.int32, (LANES, 2), 1)
    P = ((rP < SLOT_DIM) == (cP == 0)).astype(jnp.float32)
    lane_lo = jax.lax.broadcasted_iota(
        jnp.int32, (HALF, LANES), 1) < SLOT_DIM
    z8 = jnp.zeros((N_HEADS, SLOT_DIM), jnp.float32)

    def spread(x):
        return jnp.where(lane_lo,
                         jnp.broadcast_to(x[:, 0:1], (HALF, LANES)),
                         jnp.broadcast_to(x[:, 1:2], (HALF, LANES)))

    h = _dot(vals, w1_ref[...], ((1,), (0,))) + b1_ref[...]
    h = 0.5 * h * (1.0 + jax.lax.erf(h / _SQRT2))
    cv = _dot(h, w2_ref[...], ((1,), (0,))) + b2_ref[...]
    cvg = cv * (addg * (1.0 / N_HEADS))

    kn = keys / jnp.maximum(
        jnp.sqrt(jnp.sum(keys * keys, axis=1, keepdims=True)), 1e-12)
    s_mem = _dot(mem * mem, P, ((1,), (0,)), _HI)
    inv_mem = 1.0 / jnp.maximum(jnp.sqrt(s_mem), 1e-12)
    mem_n = mem * spread(inv_mem)

    kn_blk = jnp.concatenate(
        [jnp.concatenate([kn, z8], axis=1),
         jnp.concatenate([z8, kn], axis=1)], axis=0)
    sim = _dot(kn_blk, mem_n, ((1,), (1,)))

    a1 = age_ref[...] + 1.0
    ab = a1 * (1.0 / (jnp.max(a1) + 1e-8))
    ab16 = jnp.concatenate(
        [jnp.broadcast_to(ab[0:1], (N_HEADS, HALF)),
         jnp.broadcast_to(ab[1:2], (N_HEADS, HALF))], axis=0)
    beta16 = jnp.concatenate([beta, beta], axis=0)
    sim = sim * beta16 + ab16

    colI = jax.lax.broadcasted_iota(jnp.int32, (2 * N_HEADS, HALF), 1)
    rowI = jax.lax.broadcasted_iota(jnp.int32, (2 * N_HEADS, HALF), 0)
    idxg = 2 * colI + (rowI >= N_HEADS).astype(jnp.int32)

    def pair(x, comb):
        y = comb(x[:N_HEADS], x[N_HEADS:])
        return jnp.concatenate([y, y], axis=0)

    work = sim
    mask = jnp.zeros((2 * N_HEADS, HALF), dtype=jnp.bool_)
    m0 = pair(jnp.max(work, axis=1, keepdims=True), jnp.maximum)
    for _ in range(TOPK):
        m = pair(jnp.max(work, axis=1, keepdims=True), jnp.maximum)
        cand = jnp.where(work == m, idxg, NUM_SLOTS)
        first = pair(jnp.min(cand, axis=1, keepdims=True), jnp.minimum)
        sel = idxg == first
        mask = jnp.logical_or(mask, sel)
        work = jnp.where(sel, -jnp.inf, work)

    wexp = jnp.where(mask, jnp.exp(sim - m0), 0.0)
    denom = pair(jnp.sum(wexp, axis=1, keepdims=True), jnp.add)
    w2 = wexp / denom
    w_out_ref[0] = w2

    er = erase * (1.0 / N_HEADS)
    z1 = jnp.zeros((N_HEADS, 1), jnp.float32)
    er16 = jnp.concatenate(
        [jnp.concatenate([er, z1], axis=0),
         jnp.concatenate([z1, er], axis=0)], axis=1)
    e_pair = _dot(w2, er16, ((0,), (0,)))
    cvg_blk = jnp.concatenate(
        [jnp.concatenate([cvg, z8], axis=1),
         jnp.concatenate([z8, cvg], axis=1)], axis=0)
    a_pack = _dot(w2, cvg_blk, ((0,), (0,)))

    new = mem - mem * spread(e_pair) + a_pack + 1e-8
    s_new = _dot(new * new, P, ((1,), (0,)))
    dec = jax.nn.sigmoid(decay_ref[...])
    scale = dec / jnp.maximum(jnp.sqrt(s_new), 1e-12)
    newmem_ref[0] = new * spread(scale)


@jax.jit
def kernel(memory, write_keys, write_vals, erase, add_gate, beta,
           W1, b1, W2, b2, decay_gate, age):
    full = lambda s: pl.BlockSpec(s, lambda b: tuple(0 for _ in s))
    grid_spec = pl.GridSpec(
        grid=(B,),
        in_specs=[
            pl.BlockSpec((1, HALF, LANES), lambda b: (b, 0, 0)),
            full((B, N_HEADS, SLOT_DIM)),
            full((B, N_HEADS, SLOT_DIM)),
            full((B, N_HEADS, 1)),
            full((B, N_HEADS, 1)),
            full((B, N_HEADS, 1)),
            full((SLOT_DIM, BOTTLENECK)),
            full((1, BOTTLENECK)),
            full((BOTTLENECK, SLOT_DIM)),
            full((1, SLOT_DIM)),
            full((HALF, 2)),
            full((2, HALF)),
        ],
        out_specs=[
            pl.BlockSpec((1, HALF, LANES), lambda b: (b, 0, 0)),
            pl.BlockSpec((1, 2 * N_HEADS, HALF), lambda b: (b, 0, 0)),
        ],
    )
    nm2, w2 = pl.pallas_call(
        _body,
        grid_spec=grid_spec,
        out_shape=[
            jax.ShapeDtypeStruct((B, HALF, LANES), jnp.float32),
            jax.ShapeDtypeStruct((B, 2 * N_HEADS, HALF), jnp.float32),
        ],
    )(memory.reshape(B, HALF, LANES), write_keys, write_vals,
      erase[..., None], add_gate[..., None], beta[..., None],
      W1, b1.reshape(1, BOTTLENECK), W2, b2.reshape(1, SLOT_DIM),
      decay_gate.reshape(HALF, 2), jnp.transpose(age.reshape(HALF, 2)))
    new_memory = nm2.reshape(B, NUM_SLOTS, SLOT_DIM)
    weights = (w2.reshape(B, 2, N_HEADS, HALF)
               .transpose(0, 2, 3, 1).reshape(B, N_HEADS, NUM_SLOTS))
    return (new_memory, weights)


# speedup vs baseline: 1.3770x; 1.0033x over previous
import jax
import jax.numpy as jnp
from jax.experimental import pallas as pl

B = 8
NUM_SLOTS = 8192
SLOT_DIM = 64
N_HEADS = 8
TOPK = 16
BOTTLENECK = 64
HALF = NUM_SLOTS // 2
LANES = 2 * SLOT_DIM

_SQRT2 = 1.4142135623730951
_HI = jax.lax.Precision.HIGHEST


def _dot(a, b, dims, precision=None):
    return jax.lax.dot_general(a, b, (dims, ((), ())), precision=precision,
                               preferred_element_type=jnp.float32)


def _body(mem_ref, keys_ref, vals_ref, erase_ref, addg_ref, beta_ref,
          w1_ref, b1_ref, w2_ref, b2_ref, decay_ref, age_ref,
          newmem_ref, w_out_ref):
    b = pl.program_id(0)
    mem = mem_ref[0]
    keys = keys_ref[b]
    vals = vals_ref[b]
    erase = erase_ref[b]
    addg = addg_ref[b]
    beta = beta_ref[b]

    rP = jax.lax.broadcasted_iota(jnp.int32, (LANES, 2), 0)
    cP = jax.lax.broadcasted_iota(jnp.int32, (LANES, 2), 1)
    P = ((rP < SLOT_DIM) == (cP == 0)).astype(jnp.float32)
    lane_lo = jax.lax.broadcasted_iota(
        jnp.int32, (HALF, LANES), 1) < SLOT_DIM
    z8 = jnp.zeros((N_HEADS, SLOT_DIM), jnp.float32)

    def spread(x):
        return jnp.where(lane_lo,
                         jnp.broadcast_to(x[:, 0:1], (HALF, LANES)),
                         jnp.broadcast_to(x[:, 1:2], (HALF, LANES)))

    h = _dot(vals, w1_ref[...], ((1,), (0,))) + b1_ref[...]
    h = 0.5 * h * (1.0 + jax.lax.erf(h / _SQRT2))
    cv = _dot(h, w2_ref[...], ((1,), (0,))) + b2_ref[...]
    cvg = cv * (addg * (1.0 / N_HEADS))

    kn = keys / jnp.maximum(
        jnp.sqrt(jnp.sum(keys * keys, axis=1, keepdims=True)), 1e-12)
    s_mem = _dot(mem * mem, P, ((1,), (0,)), _HI)
    n_mem = jnp.maximum(jnp.sqrt(s_mem), 1e-12)
    mem_n = mem / spread(n_mem)

    kn_blk = jnp.concatenate(
        [jnp.concatenate([kn, z8], axis=1),
         jnp.concatenate([z8, kn], axis=1)], axis=0)
    sim = _dot(kn_blk, mem_n, ((1,), (1,)))

    a1 = age_ref[...] + 1.0
    ab = a1 * (1.0 / (jnp.max(a1) + 1e-8))
    ab16 = jnp.concatenate(
        [jnp.broadcast_to(ab[0:1], (N_HEADS, HALF)),
         jnp.broadcast_to(ab[1:2], (N_HEADS, HALF))], axis=0)
    beta16 = jnp.concatenate([beta, beta], axis=0)
    sim = sim * beta16 + ab16

    colI = jax.lax.broadcasted_iota(jnp.int32, (2 * N_HEADS, HALF), 1)
    rowI = jax.lax.broadcasted_iota(jnp.int32, (2 * N_HEADS, HALF), 0)
    idxg = 2 * colI + (rowI >= N_HEADS).astype(jnp.int32)

    def pair(x, comb):
        y = comb(x[:N_HEADS], x[N_HEADS:])
        return jnp.concatenate([y, y], axis=0)

    work = sim
    mask = jnp.zeros((2 * N_HEADS, HALF), dtype=jnp.bool_)
    m0 = pair(jnp.max(work, axis=1, keepdims=True), jnp.maximum)
    for _ in range(TOPK):
        m = pair(jnp.max(work, axis=1, keepdims=True), jnp.maximum)
        cand = jnp.where(work == m, idxg, NUM_SLOTS)
        first = pair(jnp.min(cand, axis=1, keepdims=True), jnp.minimum)
        sel = idxg == first
        mask = jnp.logical_or(mask, sel)
        work = jnp.where(sel, -jnp.inf, work)

    wexp = jnp.where(mask, jnp.exp(sim - m0), 0.0)
    denom = pair(jnp.sum(wexp, axis=1, keepdims=True), jnp.add)
    w2 = wexp / denom
    w_out_ref[0] = w2

    er = erase * (1.0 / N_HEADS)
    z1 = jnp.zeros((N_HEADS, 1), jnp.float32)
    er16 = jnp.concatenate(
        [jnp.concatenate([er, z1], axis=0),
         jnp.concatenate([z1, er], axis=0)], axis=1)
    e_pair = _dot(w2, er16, ((0,), (0,)))
    cvg_blk = jnp.concatenate(
        [jnp.concatenate([cvg, z8], axis=1),
         jnp.concatenate([z8, cvg], axis=1)], axis=0)
    a_pack = _dot(w2, cvg_blk, ((0,), (0,)))

    new = mem - mem * spread(e_pair) + a_pack + 1e-8
    s_new = _dot(new * new, P, ((1,), (0,)))
    dec = jax.nn.sigmoid(decay_ref[...])
    scale = dec / jnp.maximum(jnp.sqrt(s_new), 1e-12)
    newmem_ref[0] = new * spread(scale)


@jax.jit
def kernel(memory, write_keys, write_vals, erase, add_gate, beta,
           W1, b1, W2, b2, decay_gate, age):
    full = lambda s: pl.BlockSpec(s, lambda b: tuple(0 for _ in s))
    grid_spec = pl.GridSpec(
        grid=(B,),
        in_specs=[
            pl.BlockSpec((1, HALF, LANES), lambda b: (b, 0, 0)),
            full((B, N_HEADS, SLOT_DIM)),
            full((B, N_HEADS, SLOT_DIM)),
            full((B, N_HEADS, 1)),
            full((B, N_HEADS, 1)),
            full((B, N_HEADS, 1)),
            full((SLOT_DIM, BOTTLENECK)),
            full((1, BOTTLENECK)),
            full((BOTTLENECK, SLOT_DIM)),
            full((1, SLOT_DIM)),
            full((HALF, 2)),
            full((2, HALF)),
        ],
        out_specs=[
            pl.BlockSpec((1, HALF, LANES), lambda b: (b, 0, 0)),
            pl.BlockSpec((1, 2 * N_HEADS, HALF), lambda b: (b, 0, 0)),
        ],
    )
    nm2, w2 = pl.pallas_call(
        _body,
        grid_spec=grid_spec,
        out_shape=[
            jax.ShapeDtypeStruct((B, HALF, LANES), jnp.float32),
            jax.ShapeDtypeStruct((B, 2 * N_HEADS, HALF), jnp.float32),
        ],
    )(memory.reshape(B, HALF, LANES), write_keys, write_vals,
      erase[..., None], add_gate[..., None], beta[..., None],
      W1, b1.reshape(1, BOTTLENECK), W2, b2.reshape(1, SLOT_DIM),
      decay_gate.reshape(HALF, 2), jnp.transpose(age.reshape(HALF, 2)))
    new_memory = nm2.reshape(B, NUM_SLOTS, SLOT_DIM)
    weights = (w2.reshape(B, 2, N_HEADS, HALF)
               .transpose(0, 2, 3, 1).reshape(B, N_HEADS, NUM_SLOTS))
    return (new_memory, weights)


# speedup vs baseline: 1.8720x; 1.3594x over previous
import jax
import jax.numpy as jnp
from jax.experimental import pallas as pl

B = 8
NUM_SLOTS = 8192
SLOT_DIM = 64
N_HEADS = 8
TOPK = 16
BOTTLENECK = 64

_SQRT2 = 1.4142135623730951
_HI = jax.lax.Precision.HIGHEST


def _dot(a, b, dims, precision=None):
    return jax.lax.dot_general(a, b, (dims, ((), ())), precision=precision,
                               preferred_element_type=jnp.float32)


def _body(mem_ref, keys_ref, vals_ref, erase_ref, addg_ref, beta_ref,
          w1_ref, b1_ref, w2_ref, b2_ref, decay_ref, age_ref,
          newmem_ref, w_out_ref):
    b = pl.program_id(0)
    mem = mem_ref[0]
    keys = keys_ref[b]
    vals = vals_ref[b]
    erase = erase_ref[b]
    addg = addg_ref[b]
    beta = beta_ref[b]

    dec_col = jnp.reshape(jax.nn.sigmoid(decay_ref[...]), (NUM_SLOTS, 1))

    h = _dot(vals, w1_ref[...], ((1,), (0,))) + b1_ref[...]
    h = 0.5 * h * (1.0 + jax.lax.erf(h / _SQRT2))
    cv = _dot(h, w2_ref[...], ((1,), (0,))) + b2_ref[...]
    cvg = cv * (addg * (1.0 / N_HEADS))

    kn = keys / jnp.maximum(
        jnp.sqrt(jnp.sum(keys * keys, axis=1, keepdims=True)), 1e-12)
    ones64 = jnp.ones((SLOT_DIM, SLOT_DIM), jnp.float32)
    s_mem = _dot(mem * mem, ones64, ((1,), (0,)), _HI)
    mem_n = mem / jnp.maximum(jnp.sqrt(s_mem), 1e-12)

    sim = _dot(kn, mem_n, ((1,), (1,)))
    a1 = age_ref[...] + 1.0
    ab = a1 * (1.0 / (jnp.max(a1) + 1e-8))
    sim = sim * beta + ab

    iota = jax.lax.broadcasted_iota(jnp.int32, (N_HEADS, NUM_SLOTS), 1)
    work = sim
    mask = jnp.zeros((N_HEADS, NUM_SLOTS), dtype=jnp.bool_)
    m0 = jnp.max(work, axis=1, keepdims=True)
    for _ in range(TOPK):
        m = jnp.max(work, axis=1, keepdims=True)
        cand = jnp.where(work == m, iota, NUM_SLOTS)
        first = jnp.min(cand, axis=1, keepdims=True)
        sel = iota == first
        mask = jnp.logical_or(mask, sel)
        work = jnp.where(sel, -jnp.inf, work)

    wexp = jnp.where(mask, jnp.exp(sim - m0), 0.0)
    w = wexp / jnp.sum(wexp, axis=1, keepdims=True)
    w_out_ref[0] = w

    e_col = _dot(w, erase * (1.0 / N_HEADS), ((0,), (0,)))
    a_mat = _dot(w, cvg, ((0,), (0,)))
    new = mem - mem * e_col + a_mat + 1e-8
    s_new = _dot(new * new, ones64, ((1,), (0,)))
    scale = dec_col / jnp.maximum(jnp.sqrt(s_new), 1e-12)
    newmem_ref[0] = new * scale


@jax.jit
def kernel(memory, write_keys, write_vals, erase, add_gate, beta,
           W1, b1, W2, b2, decay_gate, age):
    full = lambda s: pl.BlockSpec(s, lambda b: tuple(0 for _ in s))
    grid_spec = pl.GridSpec(
        grid=(B,),
        in_specs=[
            pl.BlockSpec((1, NUM_SLOTS, SLOT_DIM), lambda b: (b, 0, 0)),
            full((B, N_HEADS, SLOT_DIM)),
            full((B, N_HEADS, SLOT_DIM)),
            full((B, N_HEADS, 1)),
            full((B, N_HEADS, 1)),
            full((B, N_HEADS, 1)),
            full((SLOT_DIM, BOTTLENECK)),
            full((1, BOTTLENECK)),
            full((BOTTLENECK, SLOT_DIM)),
            full((1, SLOT_DIM)),
            full((1, NUM_SLOTS)),
            full((1, NUM_SLOTS)),
        ],
        out_specs=[
            pl.BlockSpec((1, NUM_SLOTS, SLOT_DIM), lambda b: (b, 0, 0)),
            pl.BlockSpec((1, N_HEADS, NUM_SLOTS), lambda b: (b, 0, 0)),
        ],
    )
    new_memory, weights = pl.pallas_call(
        _body,
        grid_spec=grid_spec,
        out_shape=[
            jax.ShapeDtypeStruct((B, NUM_SLOTS, SLOT_DIM), jnp.float32),
            jax.ShapeDtypeStruct((B, N_HEADS, NUM_SLOTS), jnp.float32),
        ],
    )(memory, write_keys, write_vals,
      erase[..., None], add_gate[..., None], beta[..., None],
      W1, b1.reshape(1, BOTTLENECK), W2, b2.reshape(1, SLOT_DIM),
      decay_gate.reshape(1, NUM_SLOTS), age)
    return (new_memory, weights)


# speedup vs baseline: 2.2058x; 1.1783x over previous
import jax
import jax.numpy as jnp
from jax.experimental import pallas as pl

B = 8
NUM_SLOTS = 8192
SLOT_DIM = 64
N_HEADS = 8
TOPK = 16
BOTTLENECK = 64

_SQRT2 = 1.4142135623730951
_HI = jax.lax.Precision.HIGHEST


def _dot(a, b, dims, precision=None):
    return jax.lax.dot_general(a, b, (dims, ((), ())), precision=precision,
                               preferred_element_type=jnp.float32)


def _body(mem_ref, keys_ref, vals_ref, erase_ref, addg_ref, beta_ref,
          w1_ref, b1_ref, w2_ref, b2_ref, decay_ref, age_ref,
          newmem_ref, w_out_ref):
    b = pl.program_id(0)
    mem = mem_ref[0]
    keys = keys_ref[b]
    vals = vals_ref[b]
    erase = erase_ref[b]
    addg = addg_ref[b]
    beta = beta_ref[b]

    h = _dot(vals, w1_ref[...], ((1,), (0,))) + b1_ref[...]
    h = 0.5 * h * (1.0 + jax.lax.erf(h / _SQRT2))
    cv = _dot(h, w2_ref[...], ((1,), (0,))) + b2_ref[...]
    cvg = cv * (addg * (1.0 / N_HEADS))

    kn = keys / jnp.maximum(
        jnp.sqrt(jnp.sum(keys * keys, axis=1, keepdims=True)), 1e-12)
    ones_row = jnp.ones((1, SLOT_DIM), jnp.float32)
    s_mem = _dot(ones_row, mem * mem, ((1,), (1,)), _HI)
    n_row = jnp.maximum(jnp.sqrt(s_mem), 1e-12)
    mem_n = mem / jnp.reshape(n_row, (NUM_SLOTS, 1))

    sim = _dot(kn, mem_n, ((1,), (1,)))
    a1 = age_ref[...] + 1.0
    ab = a1 * (1.0 / (jnp.max(a1) + 1e-8))
    sim = sim * beta + ab

    iota = jax.lax.broadcasted_iota(jnp.int32, (N_HEADS, NUM_SLOTS), 1)
    work = sim
    mask = jnp.zeros((N_HEADS, NUM_SLOTS), dtype=jnp.bool_)
    m0 = jnp.max(work, axis=1, keepdims=True)
    for _ in range(TOPK):
        m = jnp.max(work, axis=1, keepdims=True)
        cand = jnp.where(work == m, iota, NUM_SLOTS)
        first = jnp.min(cand, axis=1, keepdims=True)
        sel = iota == first
        mask = jnp.logical_or(mask, sel)
        work = jnp.where(sel, -jnp.inf, work)

    wexp = jnp.where(mask, jnp.exp(sim - m0), 0.0)
    w = wexp / jnp.sum(wexp, axis=1, keepdims=True)
    w_out_ref[0] = w

    e_col = _dot(w, erase * (1.0 / N_HEADS), ((0,), (0,)))
    a_mat = _dot(w, cvg, ((0,), (0,)))
    new = mem - mem * e_col + a_mat + 1e-8
    s_new = _dot(ones_row, new * new, ((1,), (1,)))
    scale_row = (jax.nn.sigmoid(decay_ref[...])
                 / jnp.maximum(jnp.sqrt(s_new), 1e-12))
    newmem_ref[0] = new * jnp.reshape(scale_row, (NUM_SLOTS, 1))


@jax.jit
def kernel(memory, write_keys, write_vals, erase, add_gate, beta,
           W1, b1, W2, b2, decay_gate, age):
    full = lambda s: pl.BlockSpec(s, lambda b: tuple(0 for _ in s))
    grid_spec = pl.GridSpec(
        grid=(B,),
        in_specs=[
            pl.BlockSpec((1, NUM_SLOTS, SLOT_DIM), lambda b: (b, 0, 0)),
            full((B, N_HEADS, SLOT_DIM)),
            full((B, N_HEADS, SLOT_DIM)),
            full((B, N_HEADS, 1)),
            full((B, N_HEADS, 1)),
            full((B, N_HEADS, 1)),
            full((SLOT_DIM, BOTTLENECK)),
            full((1, BOTTLENECK)),
            full((BOTTLENECK, SLOT_DIM)),
            full((1, SLOT_DIM)),
            full((1, NUM_SLOTS)),
            full((1, NUM_SLOTS)),
        ],
        out_specs=[
            pl.BlockSpec((1, NUM_SLOTS, SLOT_DIM), lambda b: (b, 0, 0)),
            pl.BlockSpec((1, N_HEADS, NUM_SLOTS), lambda b: (b, 0, 0)),
        ],
    )
    new_memory, weights = pl.pallas_call(
        _body,
        grid_spec=grid_spec,
        out_shape=[
            jax.ShapeDtypeStruct((B, NUM_SLOTS, SLOT_DIM), jnp.float32),
            jax.ShapeDtypeStruct((B, N_HEADS, NUM_SLOTS), jnp.float32),
        ],
    )(memory, write_keys, write_vals,
      erase[..., None], add_gate[..., None], beta[..., None],
      W1, b1.reshape(1, BOTTLENECK), W2, b2.reshape(1, SLOT_DIM),
      decay_gate.reshape(1, NUM_SLOTS), age)
    return (new_memory, weights)
